# baseline (speedup 1.0000x reference)
import functools

import jax
import jax.numpy as jnp
from jax import lax
from jax.experimental import pallas as pl
from jax.experimental.pallas import tpu as pltpu
from jax.experimental.pallas import tpu_sc as plsc

N_NODES = 10000
N_PAD = 10240
N_EDGES = 320000
D = 128
G = 16

NC = 2
NS = 16
NW = NC * NS
EPT = N_EDGES // NW
KW = 128
EPT_PAD = 10240
EC = EPT_PAD // KW
IB = 8
NB = 2
ROWS_PT = N_PAD // NS

R = 512
GRID = N_PAD // R


def _deg_body(dst_hbm, out_hbm, dstv, onesv, zv, acc):
    c = lax.axis_index("c")
    s = lax.axis_index("s")
    w = s * NC + c

    def _fill_z(i, _):
        zv[pl.ds(i * 16, 16)] = jnp.zeros((16,), jnp.float32)
        return 0

    lax.fori_loop(0, ROWS_PT // 16, _fill_z, 0)

    def _fill_o(i, _):
        onesv[pl.ds(i * 16, 16)] = jnp.ones((16,), jnp.float32)
        return 0

    lax.fori_loop(0, KW // 16, _fill_o, 0)

    pltpu.sync_copy(zv, acc.at[pl.ds(s * ROWS_PT, ROWS_PT)])
    pltpu.sync_copy(dst_hbm.at[w], dstv)
    plsc.subcore_barrier()

    def _scatter(j, _):
        pltpu.sync_copy(onesv, acc.at[dstv.at[j]], add=True)
        return 0

    lax.fori_loop(0, EC, _scatter, 0)
    plsc.subcore_barrier()
    pltpu.sync_copy(acc.at[pl.ds(s * ROWS_PT, ROWS_PT)],
                    out_hbm.at[c, pl.ds(s * ROWS_PT, ROWS_PT)])


@functools.cache
def _deg_kernel():
    return pl.kernel(
        _deg_body,
        out_type=jax.ShapeDtypeStruct((NC, N_PAD), jnp.float32),
        mesh=plsc.VectorSubcoreMesh(core_axis_name="c", subcore_axis_name="s",
                                    num_cores=NC, num_subcores=NS),
        scratch_types=[
            pltpu.VMEM((EC, KW), jnp.int32),
            pltpu.VMEM((KW,), jnp.float32),
            pltpu.VMEM((ROWS_PT,), jnp.float32),
            pltpu.VMEM_SHARED((N_PAD,), jnp.float32),
        ],
    )


def _agg_body(y_hbm, src_hbm, dst_hbm, out_hbm, srcv, dstv,
              rows0, rows1, acc, gs0, gs1, ss0, ss1):
    c = lax.axis_index("c")
    s = lax.axis_index("s")
    w = s * NC + c

    def _fill_z(i, _):
        rows0[i // 8, pl.ds((i % 8) * 16, 16)] = jnp.zeros((16,), jnp.float32)
        return 0

    lax.fori_loop(0, (KW * D) // 16, _fill_z, 0)
    for t in range(ROWS_PT // KW):
        pltpu.sync_copy(rows0, acc.at[pl.ds(s * ROWS_PT + t * KW, KW)])
    plsc.subcore_barrier()

    rows = (rows0, rows1)
    gsem = (gs0, gs1)
    ssem = (ss0, ss1)

    def _blk(b, _):
        pltpu.sync_copy(src_hbm.at[w, pl.ds(b * IB, IB)], srcv)
        pltpu.sync_copy(dst_hbm.at[w, pl.ds(b * IB, IB)], dstv)
        gh = [None] * IB
        sh = [None] * IB
        gh[0] = pltpu.async_copy(y_hbm.at[srcv.at[0]], rows[0], gsem[0])
        for j in range(IB):
            if j + 1 < IB:
                if j - 1 >= 0:
                    sh[j - 1].wait()
                p = (j + 1) % NB
                gh[j + 1] = pltpu.async_copy(y_hbm.at[srcv.at[j + 1]],
                                             rows[p], gsem[p])
            gh[j].wait()
            sh[j] = pltpu.async_copy(rows[j % NB], acc.at[dstv.at[j]],
                                     ssem[j % NB], add=True)
        for j in range(IB - NB, IB):
            sh[j].wait()
        return 0

    lax.fori_loop(0, EC // IB, _blk, 0)
    plsc.subcore_barrier()
    pltpu.sync_copy(acc.at[pl.ds(s * ROWS_PT, ROWS_PT)],
                    out_hbm.at[c, pl.ds(s * ROWS_PT, ROWS_PT)])


@functools.cache
def _agg_kernel():
    return pl.kernel(
        _agg_body,
        out_type=jax.ShapeDtypeStruct((NC, N_PAD, D), jnp.float32),
        mesh=plsc.VectorSubcoreMesh(core_axis_name="c", subcore_axis_name="s",
                                    num_cores=NC, num_subcores=NS),
        scratch_types=(
            [pltpu.VMEM((IB, KW), jnp.int32)] * 2
            + [pltpu.VMEM((KW, D), jnp.float32)] * NB
            + [pltpu.VMEM_SHARED((N_PAD, D), jnp.float32)]
            + [pltpu.SemaphoreType.DMA] * (2 * NB)
        ),
    )


def _pre_body(degp_ref, x_ref, w_ref, y_ref):
    p = degp_ref[...]
    d = lax.rsqrt(1.0 + p[0] + p[1])
    y_ref[...] = jnp.dot(x_ref[...], w_ref[...],
                         preferred_element_type=jnp.float32) * d


def _tc_pre(degp3, x, W1):
    return pl.pallas_call(
        _pre_body,
        grid=(GRID,),
        in_specs=[
            pl.BlockSpec((NC, R, 1), lambda i: (0, i, 0)),
            pl.BlockSpec((R, D), lambda i: (i, 0)),
            pl.BlockSpec((D, D), lambda i: (0, 0)),
        ],
        out_specs=pl.BlockSpec((R, D), lambda i: (i, 0)),
        out_shape=jax.ShapeDtypeStruct((N_PAD, D), jnp.float32),
    )(degp3, x, W1)


def _mid_body(degp_ref, s_ref, y_ref, b_ref, w_ref, o_ref):
    p = degp_ref[...]
    d = lax.rsqrt(1.0 + p[0] + p[1])
    sm = s_ref[...]
    h = jnp.maximum(d * (sm[0] + sm[1] + y_ref[...]) + b_ref[...], 0.0)
    o_ref[...] = jnp.dot(h, w_ref[...],
                         preferred_element_type=jnp.float32) * d


def _tc_mid(degp3, s1, y1, b1, W2):
    return pl.pallas_call(
        _mid_body,
        grid=(GRID,),
        in_specs=[
            pl.BlockSpec((NC, R, 1), lambda i: (0, i, 0)),
            pl.BlockSpec((NC, R, D), lambda i: (0, i, 0)),
            pl.BlockSpec((R, D), lambda i: (i, 0)),
            pl.BlockSpec((1, D), lambda i: (0, 0)),
            pl.BlockSpec((D, D), lambda i: (0, 0)),
        ],
        out_specs=pl.BlockSpec((R, D), lambda i: (i, 0)),
        out_shape=jax.ShapeDtypeStruct((N_PAD, D), jnp.float32),
    )(degp3, s1, y1, b1, W2)


def _post_body(degp_ref, s_ref, y_ref, b_ref, bt_ref, o_ref, acc, cnt):
    i = pl.program_id(0)

    @pl.when(i == 0)
    def _init():
        acc[...] = jnp.zeros_like(acc)
        cnt[...] = jnp.zeros_like(cnt)

    p = degp_ref[...]
    d = lax.rsqrt(1.0 + p[0] + p[1])
    sm = s_ref[...]
    h = jnp.maximum(d * (sm[0] + sm[1] + y_ref[...]) + b_ref[...], 0.0)
    bt = bt_ref[...]
    oh = (lax.broadcasted_iota(jnp.int32, (G, R), 0) == bt).astype(jnp.float32)
    acc[...] += jnp.dot(oh, h, preferred_element_type=jnp.float32)
    cnt[...] += jnp.sum(oh, axis=1, keepdims=True)

    @pl.when(i == GRID - 1)
    def _fin():
        o_ref[...] = acc[...] / jnp.maximum(cnt[...], 1.0)


def _tc_post(degp3, s2, y2, b2, batch_row):
    return pl.pallas_call(
        _post_body,
        grid=(GRID,),
        in_specs=[
            pl.BlockSpec((NC, R, 1), lambda i: (0, i, 0)),
            pl.BlockSpec((NC, R, D), lambda i: (0, i, 0)),
            pl.BlockSpec((R, D), lambda i: (i, 0)),
            pl.BlockSpec((1, D), lambda i: (0, 0)),
            pl.BlockSpec((1, R), lambda i: (0, i)),
        ],
        out_specs=pl.BlockSpec((G, D), lambda i: (0, 0)),
        out_shape=jax.ShapeDtypeStruct((G, D), jnp.float32),
        scratch_shapes=[
            pltpu.VMEM((G, D), jnp.float32),
            pltpu.VMEM((G, 1), jnp.float32),
        ],
    )(degp3, s2, y2, b2, batch_row)


def kernel(x, edge_index, batch, W1, b1, W2, b2):
    pad_ids = N_NODES + jnp.arange(EPT_PAD - EPT, dtype=jnp.int32)[None, :]
    pad_blk = jnp.broadcast_to(pad_ids, (NW, EPT_PAD - EPT))
    src3 = jnp.concatenate(
        [edge_index[0].reshape(NW, EPT), pad_blk], axis=1).reshape(NW, EC, KW)
    dst3 = jnp.concatenate(
        [edge_index[1].reshape(NW, EPT), pad_blk], axis=1).reshape(NW, EC, KW)

    x_pad = jnp.pad(x, ((0, N_PAD - N_NODES), (0, 0)))
    batch_row = jnp.pad(batch, (0, N_PAD - N_NODES),
                        constant_values=G).reshape(1, N_PAD)
    b1r = b1.reshape(1, D)
    b2r = b2.reshape(1, D)

    degp = _deg_kernel()(dst3)
    degp3 = degp.reshape(NC, N_PAD, 1)
    y1 = _tc_pre(degp3, x_pad, W1)
    s1 = _agg_kernel()(y1, src3, dst3)
    y2 = _tc_mid(degp3, s1, y1, b1r, W2)
    s2 = _agg_kernel()(y2, src3, dst3)
    return _tc_post(degp3, s2, y2, b2r, batch_row)


# speedup vs baseline: 1.0131x; 1.0131x over previous
import functools

import jax
import jax.numpy as jnp
from jax import lax
from jax.experimental import pallas as pl
from jax.experimental.pallas import tpu as pltpu
from jax.experimental.pallas import tpu_sc as plsc

N_NODES = 10000
N_PAD = 10240
N_EDGES = 320000
D = 128
G = 16

NC = 2
NS = 16
NW = NC * NS
EPT = N_EDGES // NW
KW = 64
EPT_PAD = 10240
EC = EPT_PAD // KW
IB = 16
NB = 4
ROWS_PT = N_PAD // NS

R = 512
GRID = N_PAD // R


def _deg_body(dst_hbm, out_hbm, dstv, onesv, zv, acc):
    c = lax.axis_index("c")
    s = lax.axis_index("s")
    w = s * NC + c

    def _fill_z(i, _):
        zv[pl.ds(i * 16, 16)] = jnp.zeros((16,), jnp.float32)
        return 0

    lax.fori_loop(0, ROWS_PT // 16, _fill_z, 0)

    def _fill_o(i, _):
        onesv[pl.ds(i * 16, 16)] = jnp.ones((16,), jnp.float32)
        return 0

    lax.fori_loop(0, KW // 16, _fill_o, 0)

    pltpu.sync_copy(zv, acc.at[pl.ds(s * ROWS_PT, ROWS_PT)])
    pltpu.sync_copy(dst_hbm.at[w], dstv)
    plsc.subcore_barrier()

    def _scatter(j, _):
        pltpu.sync_copy(onesv, acc.at[dstv.at[j]], add=True)
        return 0

    lax.fori_loop(0, EC, _scatter, 0)
    plsc.subcore_barrier()
    pltpu.sync_copy(acc.at[pl.ds(s * ROWS_PT, ROWS_PT)],
                    out_hbm.at[c, pl.ds(s * ROWS_PT, ROWS_PT)])


@functools.cache
def _deg_kernel():
    return pl.kernel(
        _deg_body,
        out_type=jax.ShapeDtypeStruct((NC, N_PAD), jnp.float32),
        mesh=plsc.VectorSubcoreMesh(core_axis_name="c", subcore_axis_name="s",
                                    num_cores=NC, num_subcores=NS),
        scratch_types=[
            pltpu.VMEM((EC, KW), jnp.int32),
            pltpu.VMEM((KW,), jnp.float32),
            pltpu.VMEM((ROWS_PT,), jnp.float32),
            pltpu.VMEM_SHARED((N_PAD,), jnp.float32),
        ],
    )


def _agg_body(y_hbm, src_hbm, dst_hbm, out_hbm, srcv, dstv,
              rows0, rows1, rows2, rows3, acc,
              gs0, gs1, gs2, gs3, ss0, ss1, ss2, ss3):
    c = lax.axis_index("c")
    s = lax.axis_index("s")
    w = s * NC + c

    def _fill_z(i, _):
        rows0[i // 8, pl.ds((i % 8) * 16, 16)] = jnp.zeros((16,), jnp.float32)
        return 0

    lax.fori_loop(0, (KW * D) // 16, _fill_z, 0)
    for t in range(ROWS_PT // KW):
        pltpu.sync_copy(rows0, acc.at[pl.ds(s * ROWS_PT + t * KW, KW)])
    plsc.subcore_barrier()

    rows = (rows0, rows1, rows2, rows3)
    gsem = (gs0, gs1, gs2, gs3)
    ssem = (ss0, ss1, ss2, ss3)

    def _blk(b, _):
        pltpu.sync_copy(src_hbm.at[w, pl.ds(b * IB, IB)], srcv)
        pltpu.sync_copy(dst_hbm.at[w, pl.ds(b * IB, IB)], dstv)
        gh = [None] * IB
        sh = [None] * IB
        gh[0] = pltpu.async_copy(y_hbm.at[srcv.at[0]], rows[0], gsem[0])
        gh[1] = pltpu.async_copy(y_hbm.at[srcv.at[1]], rows[1], gsem[1])
        for j in range(IB):
            if j + 2 < IB:
                if j - 2 >= 0:
                    sh[j - 2].wait()
                p = (j + 2) % NB
                gh[j + 2] = pltpu.async_copy(y_hbm.at[srcv.at[j + 2]],
                                             rows[p], gsem[p])
            gh[j].wait()
            sh[j] = pltpu.async_copy(rows[j % NB], acc.at[dstv.at[j]],
                                     ssem[j % NB], add=True)
        for j in range(IB - NB, IB):
            sh[j].wait()
        return 0

    lax.fori_loop(0, EC // IB, _blk, 0)
    plsc.subcore_barrier()
    pltpu.sync_copy(acc.at[pl.ds(s * ROWS_PT, ROWS_PT)],
                    out_hbm.at[c, pl.ds(s * ROWS_PT, ROWS_PT)])


@functools.cache
def _agg_kernel():
    return pl.kernel(
        _agg_body,
        out_type=jax.ShapeDtypeStruct((NC, N_PAD, D), jnp.float32),
        mesh=plsc.VectorSubcoreMesh(core_axis_name="c", subcore_axis_name="s",
                                    num_cores=NC, num_subcores=NS),
        scratch_types=(
            [pltpu.VMEM((IB, KW), jnp.int32)] * 2
            + [pltpu.VMEM((KW, D), jnp.float32)] * NB
            + [pltpu.VMEM_SHARED((N_PAD, D), jnp.float32)]
            + [pltpu.SemaphoreType.DMA] * (2 * NB)
        ),
    )


def _pre_body(degp_ref, x_ref, w_ref, y_ref):
    p = degp_ref[...]
    d = lax.rsqrt(1.0 + p[0] + p[1])
    y_ref[...] = jnp.dot(x_ref[...], w_ref[...],
                         preferred_element_type=jnp.float32) * d


def _tc_pre(degp3, x, W1):
    return pl.pallas_call(
        _pre_body,
        grid=(GRID,),
        in_specs=[
            pl.BlockSpec((NC, R, 1), lambda i: (0, i, 0)),
            pl.BlockSpec((R, D), lambda i: (i, 0)),
            pl.BlockSpec((D, D), lambda i: (0, 0)),
        ],
        out_specs=pl.BlockSpec((R, D), lambda i: (i, 0)),
        out_shape=jax.ShapeDtypeStruct((N_PAD, D), jnp.float32),
    )(degp3, x, W1)


def _mid_body(degp_ref, s_ref, y_ref, b_ref, w_ref, o_ref):
    p = degp_ref[...]
    d = lax.rsqrt(1.0 + p[0] + p[1])
    sm = s_ref[...]
    h = jnp.maximum(d * (sm[0] + sm[1] + y_ref[...]) + b_ref[...], 0.0)
    o_ref[...] = jnp.dot(h, w_ref[...],
                         preferred_element_type=jnp.float32) * d


def _tc_mid(degp3, s1, y1, b1, W2):
    return pl.pallas_call(
        _mid_body,
        grid=(GRID,),
        in_specs=[
            pl.BlockSpec((NC, R, 1), lambda i: (0, i, 0)),
            pl.BlockSpec((NC, R, D), lambda i: (0, i, 0)),
            pl.BlockSpec((R, D), lambda i: (i, 0)),
            pl.BlockSpec((1, D), lambda i: (0, 0)),
            pl.BlockSpec((D, D), lambda i: (0, 0)),
        ],
        out_specs=pl.BlockSpec((R, D), lambda i: (i, 0)),
        out_shape=jax.ShapeDtypeStruct((N_PAD, D), jnp.float32),
    )(degp3, s1, y1, b1, W2)


def _post_body(degp_ref, s_ref, y_ref, b_ref, bt_ref, o_ref, acc, cnt):
    i = pl.program_id(0)

    @pl.when(i == 0)
    def _init():
        acc[...] = jnp.zeros_like(acc)
        cnt[...] = jnp.zeros_like(cnt)

    p = degp_ref[...]
    d = lax.rsqrt(1.0 + p[0] + p[1])
    sm = s_ref[...]
    h = jnp.maximum(d * (sm[0] + sm[1] + y_ref[...]) + b_ref[...], 0.0)
    bt = bt_ref[...]
    oh = (lax.broadcasted_iota(jnp.int32, (G, R), 0) == bt).astype(jnp.float32)
    acc[...] += jnp.dot(oh, h, preferred_element_type=jnp.float32)
    cnt[...] += jnp.sum(oh, axis=1, keepdims=True)

    @pl.when(i == GRID - 1)
    def _fin():
        o_ref[...] = acc[...] / jnp.maximum(cnt[...], 1.0)


def _tc_post(degp3, s2, y2, b2, batch_row):
    return pl.pallas_call(
        _post_body,
        grid=(GRID,),
        in_specs=[
            pl.BlockSpec((NC, R, 1), lambda i: (0, i, 0)),
            pl.BlockSpec((NC, R, D), lambda i: (0, i, 0)),
            pl.BlockSpec((R, D), lambda i: (i, 0)),
            pl.BlockSpec((1, D), lambda i: (0, 0)),
            pl.BlockSpec((1, R), lambda i: (0, i)),
        ],
        out_specs=pl.BlockSpec((G, D), lambda i: (0, 0)),
        out_shape=jax.ShapeDtypeStruct((G, D), jnp.float32),
        scratch_shapes=[
            pltpu.VMEM((G, D), jnp.float32),
            pltpu.VMEM((G, 1), jnp.float32),
        ],
    )(degp3, s2, y2, b2, batch_row)


def kernel(x, edge_index, batch, W1, b1, W2, b2):
    pad_ids = N_NODES + jnp.arange(EPT_PAD - EPT, dtype=jnp.int32)[None, :]
    pad_blk = jnp.broadcast_to(pad_ids, (NW, EPT_PAD - EPT))
    src3 = jnp.concatenate(
        [edge_index[0].reshape(NW, EPT), pad_blk], axis=1).reshape(NW, EC, KW)
    dst3 = jnp.concatenate(
        [edge_index[1].reshape(NW, EPT), pad_blk], axis=1).reshape(NW, EC, KW)

    x_pad = jnp.pad(x, ((0, N_PAD - N_NODES), (0, 0)))
    batch_row = jnp.pad(batch, (0, N_PAD - N_NODES),
                        constant_values=G).reshape(1, N_PAD)
    b1r = b1.reshape(1, D)
    b2r = b2.reshape(1, D)

    degp = _deg_kernel()(dst3)
    degp3 = degp.reshape(NC, N_PAD, 1)
    y1 = _tc_pre(degp3, x_pad, W1)
    s1 = _agg_kernel()(y1, src3, dst3)
    y2 = _tc_mid(degp3, s1, y1, b1r, W2)
    s2 = _agg_kernel()(y2, src3, dst3)
    return _tc_post(degp3, s2, y2, b2r, batch_row)


# speedup vs baseline: 1.0405x; 1.0271x over previous
import functools

import jax
import jax.numpy as jnp
from jax import lax
from jax.experimental import pallas as pl
from jax.experimental.pallas import tpu as pltpu
from jax.experimental.pallas import tpu_sc as plsc

N_NODES = 10000
N_PAD = 10240
N_EDGES = 320000
D = 128
G = 16

NC = 2
NS = 16
NW = NC * NS
EPT = N_EDGES // NW
KW = 64
EPT_PAD = 10240
EC = EPT_PAD // KW
IB = 16
NB = 4
ROWS_PT = N_PAD // NS

R = 512
GRID = N_PAD // R


def _deg_body(dst_hbm, out_hbm, dstv, onesv, zv, acc, dsem):
    c = lax.axis_index("c")
    s = lax.axis_index("s")
    w = s * NC + c

    def _fill_z(i, _):
        zv[pl.ds(i * 16, 16)] = jnp.zeros((16,), jnp.float32)
        return 0

    lax.fori_loop(0, ROWS_PT // 16, _fill_z, 0)

    def _fill_o(i, _):
        onesv[pl.ds(i * 16, 16)] = jnp.ones((16,), jnp.float32)
        return 0

    lax.fori_loop(0, KW // 16, _fill_o, 0)

    pltpu.sync_copy(zv, acc.at[pl.ds(s * ROWS_PT, ROWS_PT)])
    pltpu.sync_copy(dst_hbm.at[w], dstv)
    plsc.subcore_barrier()

    def _scatter(b, _):
        hs = [pltpu.async_copy(onesv, acc.at[dstv.at[b * 16 + j]], dsem,
                               add=True) for j in range(16)]
        for h in hs:
            h.wait()
        return 0

    lax.fori_loop(0, EC // 16, _scatter, 0)
    plsc.subcore_barrier()
    pltpu.sync_copy(acc.at[pl.ds(s * ROWS_PT, ROWS_PT)],
                    out_hbm.at[c, pl.ds(s * ROWS_PT, ROWS_PT)])


@functools.cache
def _deg_kernel():
    return pl.kernel(
        _deg_body,
        out_type=jax.ShapeDtypeStruct((NC, N_PAD), jnp.float32),
        mesh=plsc.VectorSubcoreMesh(core_axis_name="c", subcore_axis_name="s",
                                    num_cores=NC, num_subcores=NS),
        scratch_types=[
            pltpu.VMEM((EC, KW), jnp.int32),
            pltpu.VMEM((KW,), jnp.float32),
            pltpu.VMEM((ROWS_PT,), jnp.float32),
            pltpu.VMEM_SHARED((N_PAD,), jnp.float32),
            pltpu.SemaphoreType.DMA,
        ],
    )


def _agg_body(y_hbm, src_hbm, dst_hbm, out_hbm, srcv, dstv,
              rows0, rows1, rows2, rows3, acc,
              gs0, gs1, gs2, gs3, ss0, ss1, ss2, ss3):
    c = lax.axis_index("c")
    s = lax.axis_index("s")
    w = s * NC + c

    def _fill_z(i, _):
        rows0[i // 8, pl.ds((i % 8) * 16, 16)] = jnp.zeros((16,), jnp.float32)
        return 0

    lax.fori_loop(0, (KW * D) // 16, _fill_z, 0)
    zh = [pltpu.async_copy(rows0, acc.at[pl.ds(s * ROWS_PT + t * KW, KW)], gs0)
          for t in range(ROWS_PT // KW)]
    for h in zh:
        h.wait()
    plsc.subcore_barrier()

    rows = (rows0, rows1, rows2, rows3)
    gsem = (gs0, gs1, gs2, gs3)
    ssem = (ss0, ss1, ss2, ss3)

    def _blk(b, _):
        pltpu.sync_copy(src_hbm.at[w, pl.ds(b * IB, IB)], srcv)
        pltpu.sync_copy(dst_hbm.at[w, pl.ds(b * IB, IB)], dstv)
        gh = [None] * IB
        sh = [None] * IB
        gh[0] = pltpu.async_copy(y_hbm.at[srcv.at[0]], rows[0], gsem[0])
        gh[1] = pltpu.async_copy(y_hbm.at[srcv.at[1]], rows[1], gsem[1])
        for j in range(IB):
            if j + 2 < IB:
                if j - 2 >= 0:
                    sh[j - 2].wait()
                p = (j + 2) % NB
                gh[j + 2] = pltpu.async_copy(y_hbm.at[srcv.at[j + 2]],
                                             rows[p], gsem[p])
            gh[j].wait()
            sh[j] = pltpu.async_copy(rows[j % NB], acc.at[dstv.at[j]],
                                     ssem[j % NB], add=True)
        for j in range(IB - NB, IB):
            sh[j].wait()
        return 0

    lax.fori_loop(0, EC // IB, _blk, 0)
    plsc.subcore_barrier()
    pltpu.sync_copy(acc.at[pl.ds(s * ROWS_PT, ROWS_PT)],
                    out_hbm.at[c, pl.ds(s * ROWS_PT, ROWS_PT)])


@functools.cache
def _agg_kernel():
    return pl.kernel(
        _agg_body,
        out_type=jax.ShapeDtypeStruct((NC, N_PAD, D), jnp.float32),
        mesh=plsc.VectorSubcoreMesh(core_axis_name="c", subcore_axis_name="s",
                                    num_cores=NC, num_subcores=NS),
        scratch_types=(
            [pltpu.VMEM((IB, KW), jnp.int32)] * 2
            + [pltpu.VMEM((KW, D), jnp.float32)] * NB
            + [pltpu.VMEM_SHARED((N_PAD, D), jnp.float32)]
            + [pltpu.SemaphoreType.DMA] * (2 * NB)
        ),
    )


def _pre_body(degp_ref, x_ref, w_ref, y_ref):
    p = degp_ref[...]
    d = lax.rsqrt(1.0 + p[0] + p[1])
    y_ref[...] = jnp.dot(x_ref[...], w_ref[...],
                         preferred_element_type=jnp.float32) * d


def _tc_pre(degp3, x, W1):
    return pl.pallas_call(
        _pre_body,
        grid=(GRID,),
        in_specs=[
            pl.BlockSpec((NC, R, 1), lambda i: (0, i, 0)),
            pl.BlockSpec((R, D), lambda i: (i, 0)),
            pl.BlockSpec((D, D), lambda i: (0, 0)),
        ],
        out_specs=pl.BlockSpec((R, D), lambda i: (i, 0)),
        out_shape=jax.ShapeDtypeStruct((N_PAD, D), jnp.float32),
    )(degp3, x, W1)


def _mid_body(degp_ref, s_ref, y_ref, b_ref, w_ref, o_ref):
    p = degp_ref[...]
    d = lax.rsqrt(1.0 + p[0] + p[1])
    sm = s_ref[...]
    h = jnp.maximum(d * (sm[0] + sm[1] + y_ref[...]) + b_ref[...], 0.0)
    o_ref[...] = jnp.dot(h, w_ref[...],
                         preferred_element_type=jnp.float32) * d


def _tc_mid(degp3, s1, y1, b1, W2):
    return pl.pallas_call(
        _mid_body,
        grid=(GRID,),
        in_specs=[
            pl.BlockSpec((NC, R, 1), lambda i: (0, i, 0)),
            pl.BlockSpec((NC, R, D), lambda i: (0, i, 0)),
            pl.BlockSpec((R, D), lambda i: (i, 0)),
            pl.BlockSpec((1, D), lambda i: (0, 0)),
            pl.BlockSpec((D, D), lambda i: (0, 0)),
        ],
        out_specs=pl.BlockSpec((R, D), lambda i: (i, 0)),
        out_shape=jax.ShapeDtypeStruct((N_PAD, D), jnp.float32),
    )(degp3, s1, y1, b1, W2)


def _post_body(degp_ref, s_ref, y_ref, b_ref, bt_ref, o_ref, acc, cnt):
    i = pl.program_id(0)

    @pl.when(i == 0)
    def _init():
        acc[...] = jnp.zeros_like(acc)
        cnt[...] = jnp.zeros_like(cnt)

    p = degp_ref[...]
    d = lax.rsqrt(1.0 + p[0] + p[1])
    sm = s_ref[...]
    h = jnp.maximum(d * (sm[0] + sm[1] + y_ref[...]) + b_ref[...], 0.0)
    bt = bt_ref[...]
    oh = (lax.broadcasted_iota(jnp.int32, (G, R), 0) == bt).astype(jnp.float32)
    acc[...] += jnp.dot(oh, h, preferred_element_type=jnp.float32)
    cnt[...] += jnp.sum(oh, axis=1, keepdims=True)

    @pl.when(i == GRID - 1)
    def _fin():
        o_ref[...] = acc[...] / jnp.maximum(cnt[...], 1.0)


def _tc_post(degp3, s2, y2, b2, batch_row):
    return pl.pallas_call(
        _post_body,
        grid=(GRID,),
        in_specs=[
            pl.BlockSpec((NC, R, 1), lambda i: (0, i, 0)),
            pl.BlockSpec((NC, R, D), lambda i: (0, i, 0)),
            pl.BlockSpec((R, D), lambda i: (i, 0)),
            pl.BlockSpec((1, D), lambda i: (0, 0)),
            pl.BlockSpec((1, R), lambda i: (0, i)),
        ],
        out_specs=pl.BlockSpec((G, D), lambda i: (0, 0)),
        out_shape=jax.ShapeDtypeStruct((G, D), jnp.float32),
        scratch_shapes=[
            pltpu.VMEM((G, D), jnp.float32),
            pltpu.VMEM((G, 1), jnp.float32),
        ],
    )(degp3, s2, y2, b2, batch_row)


def kernel(x, edge_index, batch, W1, b1, W2, b2):
    pad_ids = N_NODES + jnp.arange(EPT_PAD - EPT, dtype=jnp.int32)[None, :]
    pad_blk = jnp.broadcast_to(pad_ids, (NW, EPT_PAD - EPT))
    src3 = jnp.concatenate(
        [edge_index[0].reshape(NW, EPT), pad_blk], axis=1).reshape(NW, EC, KW)
    dst3 = jnp.concatenate(
        [edge_index[1].reshape(NW, EPT), pad_blk], axis=1).reshape(NW, EC, KW)

    x_pad = jnp.pad(x, ((0, N_PAD - N_NODES), (0, 0)))
    batch_row = jnp.pad(batch, (0, N_PAD - N_NODES),
                        constant_values=G).reshape(1, N_PAD)
    b1r = b1.reshape(1, D)
    b2r = b2.reshape(1, D)

    degp = _deg_kernel()(dst3)
    degp3 = degp.reshape(NC, N_PAD, 1)
    y1 = _tc_pre(degp3, x_pad, W1)
    s1 = _agg_kernel()(y1, src3, dst3)
    y2 = _tc_mid(degp3, s1, y1, b1r, W2)
    s2 = _agg_kernel()(y2, src3, dst3)
    return _tc_post(degp3, s2, y2, b2r, batch_row)


# speedup vs baseline: 1.1100x; 1.0668x over previous
import functools

import jax
import jax.numpy as jnp
from jax import lax
from jax.experimental import pallas as pl
from jax.experimental.pallas import tpu as pltpu
from jax.experimental.pallas import tpu_sc as plsc

N_NODES = 10000
N_PAD = 10240
N_EDGES = 320000
D = 128
G = 16

NC = 2
NS = 16
NW = NC * NS
EPT = N_EDGES // NW
KW = 64
EPT_PAD = 10240
EC = EPT_PAD // KW
IB = 16
NB = 4
ROWS_PT = N_PAD // NS

R = 512
GRID = N_PAD // R


def _deg_body(dst_hbm, out_hbm, dstv, onesv, zv, acc, dsem):
    c = lax.axis_index("c")
    s = lax.axis_index("s")
    w = s * NC + c

    def _fill_z(i, _):
        zv[pl.ds(i * 16, 16)] = jnp.zeros((16,), jnp.float32)
        return 0

    lax.fori_loop(0, ROWS_PT // 16, _fill_z, 0)

    def _fill_o(i, _):
        onesv[pl.ds(i * 16, 16)] = jnp.ones((16,), jnp.float32)
        return 0

    lax.fori_loop(0, KW // 16, _fill_o, 0)

    pltpu.sync_copy(zv, acc.at[pl.ds(s * ROWS_PT, ROWS_PT)])
    pltpu.sync_copy(dst_hbm.at[w], dstv)
    plsc.subcore_barrier()

    def _scatter(b, _):
        hs = [pltpu.async_copy(onesv, acc.at[dstv.at[b * 16 + j]], dsem,
                               add=True) for j in range(16)]
        for h in hs:
            h.wait()
        return 0

    lax.fori_loop(0, EC // 16, _scatter, 0)
    plsc.subcore_barrier()
    pltpu.sync_copy(acc.at[pl.ds(s * ROWS_PT, ROWS_PT)],
                    out_hbm.at[c, pl.ds(s * ROWS_PT, ROWS_PT)])


@functools.cache
def _deg_kernel():
    return pl.kernel(
        _deg_body,
        out_type=jax.ShapeDtypeStruct((NC, N_PAD), jnp.float32),
        mesh=plsc.VectorSubcoreMesh(core_axis_name="c", subcore_axis_name="s",
                                    num_cores=NC, num_subcores=NS),
        scratch_types=[
            pltpu.VMEM((EC, KW), jnp.int32),
            pltpu.VMEM((KW,), jnp.float32),
            pltpu.VMEM((ROWS_PT,), jnp.float32),
            pltpu.VMEM_SHARED((N_PAD,), jnp.float32),
            pltpu.SemaphoreType.DMA,
        ],
    )


def _agg_body(y_hbm, src_hbm, dst_hbm, out_hbm, srcv, dstv,
              rows0, rows1, rows2, rows3, acc,
              gs0, gs1, gs2, gs3, ss0, ss1, ss2, ss3, is0, is1):
    c = lax.axis_index("c")
    s = lax.axis_index("s")
    w = s * NC + c

    def _fill_z(i, _):
        rows0[i // 8, pl.ds((i % 8) * 16, 16)] = jnp.zeros((16,), jnp.float32)
        return 0

    lax.fori_loop(0, (KW * D) // 16, _fill_z, 0)
    zh = [pltpu.async_copy(rows0, acc.at[pl.ds(s * ROWS_PT + t * KW, KW)], gs0)
          for t in range(ROWS_PT // KW)]
    for h in zh:
        h.wait()
    plsc.subcore_barrier()

    rows = (rows0, rows1, rows2, rows3)
    gsem = (gs0, gs1, gs2, gs3)
    ssem = (ss0, ss1, ss2, ss3)
    NBLK = EC // IB

    pltpu.async_copy(src_hbm.at[w, pl.ds(0, IB)], srcv.at[0], is0)
    pltpu.async_copy(dst_hbm.at[w, pl.ds(0, IB)], dstv.at[0], is1)

    def _blk(b, _):
        p = b % 2
        pltpu.make_async_copy(src_hbm.at[w, pl.ds(0, IB)],
                              srcv.at[0], is0).wait()
        pltpu.make_async_copy(dst_hbm.at[w, pl.ds(0, IB)],
                              dstv.at[0], is1).wait()
        nb = ((b + 1) % NBLK) * IB
        pltpu.async_copy(src_hbm.at[w, pl.ds(nb, IB)], srcv.at[1 - p], is0)
        pltpu.async_copy(dst_hbm.at[w, pl.ds(nb, IB)], dstv.at[1 - p], is1)
        gh = [None] * IB
        sh = [None] * IB
        gh[0] = pltpu.async_copy(y_hbm.at[srcv.at[p, 0]], rows[0], gsem[0])
        gh[1] = pltpu.async_copy(y_hbm.at[srcv.at[p, 1]], rows[1], gsem[1])
        for j in range(IB):
            if j + 2 < IB:
                if j - 2 >= 0:
                    sh[j - 2].wait()
                q = (j + 2) % NB
                gh[j + 2] = pltpu.async_copy(y_hbm.at[srcv.at[p, j + 2]],
                                             rows[q], gsem[q])
            gh[j].wait()
            sh[j] = pltpu.async_copy(rows[j % NB], acc.at[dstv.at[p, j]],
                                     ssem[j % NB], add=True)
        for j in range(IB - NB, IB):
            sh[j].wait()
        return 0

    lax.fori_loop(0, NBLK, _blk, 0)
    pltpu.make_async_copy(src_hbm.at[w, pl.ds(0, IB)], srcv.at[0], is0).wait()
    pltpu.make_async_copy(dst_hbm.at[w, pl.ds(0, IB)], dstv.at[0], is1).wait()
    plsc.subcore_barrier()
    pltpu.sync_copy(acc.at[pl.ds(s * ROWS_PT, ROWS_PT)],
                    out_hbm.at[c, pl.ds(s * ROWS_PT, ROWS_PT)])


@functools.cache
def _agg_kernel():
    return pl.kernel(
        _agg_body,
        out_type=jax.ShapeDtypeStruct((NC, N_PAD, D), jnp.float32),
        mesh=plsc.VectorSubcoreMesh(core_axis_name="c", subcore_axis_name="s",
                                    num_cores=NC, num_subcores=NS),
        scratch_types=(
            [pltpu.VMEM((2, IB, KW), jnp.int32)] * 2
            + [pltpu.VMEM((KW, D), jnp.float32)] * NB
            + [pltpu.VMEM_SHARED((N_PAD, D), jnp.float32)]
            + [pltpu.SemaphoreType.DMA] * (2 * NB + 2)
        ),
    )


def _pre_body(degp_ref, x_ref, w_ref, y_ref):
    p = degp_ref[...]
    d = lax.rsqrt(1.0 + p[0] + p[1])
    y_ref[...] = jnp.dot(x_ref[...], w_ref[...],
                         preferred_element_type=jnp.float32) * d


def _tc_pre(degp3, x, W1):
    return pl.pallas_call(
        _pre_body,
        grid=(GRID,),
        in_specs=[
            pl.BlockSpec((NC, R, 1), lambda i: (0, i, 0)),
            pl.BlockSpec((R, D), lambda i: (i, 0)),
            pl.BlockSpec((D, D), lambda i: (0, 0)),
        ],
        out_specs=pl.BlockSpec((R, D), lambda i: (i, 0)),
        out_shape=jax.ShapeDtypeStruct((N_PAD, D), jnp.float32),
    )(degp3, x, W1)


def _mid_body(degp_ref, s_ref, y_ref, b_ref, w_ref, o_ref):
    p = degp_ref[...]
    d = lax.rsqrt(1.0 + p[0] + p[1])
    sm = s_ref[...]
    h = jnp.maximum(d * (sm[0] + sm[1] + y_ref[...]) + b_ref[...], 0.0)
    o_ref[...] = jnp.dot(h, w_ref[...],
                         preferred_element_type=jnp.float32) * d


def _tc_mid(degp3, s1, y1, b1, W2):
    return pl.pallas_call(
        _mid_body,
        grid=(GRID,),
        in_specs=[
            pl.BlockSpec((NC, R, 1), lambda i: (0, i, 0)),
            pl.BlockSpec((NC, R, D), lambda i: (0, i, 0)),
            pl.BlockSpec((R, D), lambda i: (i, 0)),
            pl.BlockSpec((1, D), lambda i: (0, 0)),
            pl.BlockSpec((D, D), lambda i: (0, 0)),
        ],
        out_specs=pl.BlockSpec((R, D), lambda i: (i, 0)),
        out_shape=jax.ShapeDtypeStruct((N_PAD, D), jnp.float32),
    )(degp3, s1, y1, b1, W2)


def _post_body(degp_ref, s_ref, y_ref, b_ref, bt_ref, o_ref, acc, cnt):
    i = pl.program_id(0)

    @pl.when(i == 0)
    def _init():
        acc[...] = jnp.zeros_like(acc)
        cnt[...] = jnp.zeros_like(cnt)

    p = degp_ref[...]
    d = lax.rsqrt(1.0 + p[0] + p[1])
    sm = s_ref[...]
    h = jnp.maximum(d * (sm[0] + sm[1] + y_ref[...]) + b_ref[...], 0.0)
    bt = bt_ref[...]
    oh = (lax.broadcasted_iota(jnp.int32, (G, R), 0) == bt).astype(jnp.float32)
    acc[...] += jnp.dot(oh, h, preferred_element_type=jnp.float32)
    cnt[...] += jnp.sum(oh, axis=1, keepdims=True)

    @pl.when(i == GRID - 1)
    def _fin():
        o_ref[...] = acc[...] / jnp.maximum(cnt[...], 1.0)


def _tc_post(degp3, s2, y2, b2, batch_row):
    return pl.pallas_call(
        _post_body,
        grid=(GRID,),
        in_specs=[
            pl.BlockSpec((NC, R, 1), lambda i: (0, i, 0)),
            pl.BlockSpec((NC, R, D), lambda i: (0, i, 0)),
            pl.BlockSpec((R, D), lambda i: (i, 0)),
            pl.BlockSpec((1, D), lambda i: (0, 0)),
            pl.BlockSpec((1, R), lambda i: (0, i)),
        ],
        out_specs=pl.BlockSpec((G, D), lambda i: (0, 0)),
        out_shape=jax.ShapeDtypeStruct((G, D), jnp.float32),
        scratch_shapes=[
            pltpu.VMEM((G, D), jnp.float32),
            pltpu.VMEM((G, 1), jnp.float32),
        ],
    )(degp3, s2, y2, b2, batch_row)


def kernel(x, edge_index, batch, W1, b1, W2, b2):
    pad_ids = N_NODES + jnp.arange(EPT_PAD - EPT, dtype=jnp.int32)[None, :]
    pad_blk = jnp.broadcast_to(pad_ids, (NW, EPT_PAD - EPT))
    src3 = jnp.concatenate(
        [edge_index[0].reshape(NW, EPT), pad_blk], axis=1).reshape(NW, EC, KW)
    dst3 = jnp.concatenate(
        [edge_index[1].reshape(NW, EPT), pad_blk], axis=1).reshape(NW, EC, KW)

    x_pad = jnp.pad(x, ((0, N_PAD - N_NODES), (0, 0)))
    batch_row = jnp.pad(batch, (0, N_PAD - N_NODES),
                        constant_values=G).reshape(1, N_PAD)
    b1r = b1.reshape(1, D)
    b2r = b2.reshape(1, D)

    degp = _deg_kernel()(dst3)
    degp3 = degp.reshape(NC, N_PAD, 1)
    y1 = _tc_pre(degp3, x_pad, W1)
    s1 = _agg_kernel()(y1, src3, dst3)
    y2 = _tc_mid(degp3, s1, y1, b1r, W2)
    s2 = _agg_kernel()(y2, src3, dst3)
    return _tc_post(degp3, s2, y2, b2r, batch_row)


# speedup vs baseline: 1.1401x; 1.0271x over previous
import functools

import jax
import jax.numpy as jnp
from jax import lax
from jax.experimental import pallas as pl
from jax.experimental.pallas import tpu as pltpu
from jax.experimental.pallas import tpu_sc as plsc

N_NODES = 10000
N_PAD = 10240
N_EDGES = 320000
D = 128
G = 16

NC = 2
NS = 16
NW = NC * NS
EPT = N_EDGES // NW
KW = 64
EPT_PAD = 10240
EC = EPT_PAD // KW
IB = 32
NB = 4
ROWS_PT = N_PAD // NS

R = 512
GRID = N_PAD // R


def _deg_body(dst_hbm, out_hbm, dstv, onesv, zv, acc, dsem):
    c = lax.axis_index("c")
    s = lax.axis_index("s")
    w = s * NC + c

    def _fill_z(i, _):
        zv[pl.ds(i * 16, 16)] = jnp.zeros((16,), jnp.float32)
        return 0

    lax.fori_loop(0, ROWS_PT // 16, _fill_z, 0)

    def _fill_o(i, _):
        onesv[pl.ds(i * 16, 16)] = jnp.ones((16,), jnp.float32)
        return 0

    lax.fori_loop(0, KW // 16, _fill_o, 0)

    pltpu.sync_copy(zv, acc.at[pl.ds(s * ROWS_PT, ROWS_PT)])
    pltpu.sync_copy(dst_hbm.at[w], dstv)
    plsc.subcore_barrier()

    def _scatter(b, _):
        hs = [pltpu.async_copy(onesv, acc.at[dstv.at[b * 16 + j]], dsem,
                               add=True) for j in range(16)]
        for h in hs:
            h.wait()
        return 0

    lax.fori_loop(0, EC // 16, _scatter, 0)
    plsc.subcore_barrier()
    pltpu.sync_copy(acc.at[pl.ds(s * ROWS_PT, ROWS_PT)],
                    out_hbm.at[c, pl.ds(s * ROWS_PT, ROWS_PT)])


@functools.cache
def _deg_kernel():
    return pl.kernel(
        _deg_body,
        out_type=jax.ShapeDtypeStruct((NC, N_PAD), jnp.float32),
        mesh=plsc.VectorSubcoreMesh(core_axis_name="c", subcore_axis_name="s",
                                    num_cores=NC, num_subcores=NS),
        scratch_types=[
            pltpu.VMEM((EC, KW), jnp.int32),
            pltpu.VMEM((KW,), jnp.float32),
            pltpu.VMEM((ROWS_PT,), jnp.float32),
            pltpu.VMEM_SHARED((N_PAD,), jnp.float32),
            pltpu.SemaphoreType.DMA,
        ],
    )


def _agg_body(y_hbm, src_hbm, dst_hbm, out_hbm, srcv, dstv,
              rows0, rows1, rows2, rows3, acc,
              gs0, gs1, gs2, gs3, ss0, ss1, ss2, ss3, is0, is1):
    c = lax.axis_index("c")
    s = lax.axis_index("s")
    w = s * NC + c

    def _fill_z(i, _):
        rows0[i // 8, pl.ds((i % 8) * 16, 16)] = jnp.zeros((16,), jnp.float32)
        return 0

    lax.fori_loop(0, (KW * D) // 16, _fill_z, 0)
    zh = [pltpu.async_copy(rows0, acc.at[pl.ds(s * ROWS_PT + t * KW, KW)], gs0)
          for t in range(ROWS_PT // KW)]
    for h in zh:
        h.wait()
    plsc.subcore_barrier()

    rows = (rows0, rows1, rows2, rows3)
    gsem = (gs0, gs1, gs2, gs3)
    ssem = (ss0, ss1, ss2, ss3)
    NBLK = EC // IB

    pltpu.async_copy(src_hbm.at[w, pl.ds(0, IB)], srcv.at[0], is0)
    pltpu.async_copy(dst_hbm.at[w, pl.ds(0, IB)], dstv.at[0], is1)

    def _blk(b, _):
        p = b % 2
        pltpu.make_async_copy(src_hbm.at[w, pl.ds(0, IB)],
                              srcv.at[0], is0).wait()
        pltpu.make_async_copy(dst_hbm.at[w, pl.ds(0, IB)],
                              dstv.at[0], is1).wait()
        nb = ((b + 1) % NBLK) * IB
        pltpu.async_copy(src_hbm.at[w, pl.ds(nb, IB)], srcv.at[1 - p], is0)
        pltpu.async_copy(dst_hbm.at[w, pl.ds(nb, IB)], dstv.at[1 - p], is1)
        gh = [None] * IB
        sh = [None] * IB
        gh[0] = pltpu.async_copy(y_hbm.at[srcv.at[p, 0]], rows[0], gsem[0])
        gh[1] = pltpu.async_copy(y_hbm.at[srcv.at[p, 1]], rows[1], gsem[1])
        for j in range(IB):
            if j + 2 < IB:
                if j - 2 >= 0:
                    sh[j - 2].wait()
                q = (j + 2) % NB
                gh[j + 2] = pltpu.async_copy(y_hbm.at[srcv.at[p, j + 2]],
                                             rows[q], gsem[q])
            gh[j].wait()
            sh[j] = pltpu.async_copy(rows[j % NB], acc.at[dstv.at[p, j]],
                                     ssem[j % NB], add=True)
        for j in range(IB - NB, IB):
            sh[j].wait()
        return 0

    lax.fori_loop(0, NBLK, _blk, 0)
    pltpu.make_async_copy(src_hbm.at[w, pl.ds(0, IB)], srcv.at[0], is0).wait()
    pltpu.make_async_copy(dst_hbm.at[w, pl.ds(0, IB)], dstv.at[0], is1).wait()
    plsc.subcore_barrier()
    pltpu.sync_copy(acc.at[pl.ds(s * ROWS_PT, ROWS_PT)],
                    out_hbm.at[c, pl.ds(s * ROWS_PT, ROWS_PT)])


@functools.cache
def _agg_kernel():
    return pl.kernel(
        _agg_body,
        out_type=jax.ShapeDtypeStruct((NC, N_PAD, D), jnp.float32),
        mesh=plsc.VectorSubcoreMesh(core_axis_name="c", subcore_axis_name="s",
                                    num_cores=NC, num_subcores=NS),
        scratch_types=(
            [pltpu.VMEM((2, IB, KW), jnp.int32)] * 2
            + [pltpu.VMEM((KW, D), jnp.float32)] * NB
            + [pltpu.VMEM_SHARED((N_PAD, D), jnp.float32)]
            + [pltpu.SemaphoreType.DMA] * (2 * NB + 2)
        ),
    )


def _pre_body(degp_ref, x_ref, w_ref, y_ref):
    p = degp_ref[...]
    d = lax.rsqrt(1.0 + p[0] + p[1])
    y_ref[...] = jnp.dot(x_ref[...], w_ref[...],
                         preferred_element_type=jnp.float32) * d


def _tc_pre(degp3, x, W1):
    return pl.pallas_call(
        _pre_body,
        grid=(GRID,),
        in_specs=[
            pl.BlockSpec((NC, R, 1), lambda i: (0, i, 0)),
            pl.BlockSpec((R, D), lambda i: (i, 0)),
            pl.BlockSpec((D, D), lambda i: (0, 0)),
        ],
        out_specs=pl.BlockSpec((R, D), lambda i: (i, 0)),
        out_shape=jax.ShapeDtypeStruct((N_PAD, D), jnp.float32),
    )(degp3, x, W1)


def _mid_body(degp_ref, s_ref, y_ref, b_ref, w_ref, o_ref):
    p = degp_ref[...]
    d = lax.rsqrt(1.0 + p[0] + p[1])
    sm = s_ref[...]
    h = jnp.maximum(d * (sm[0] + sm[1] + y_ref[...]) + b_ref[...], 0.0)
    o_ref[...] = jnp.dot(h, w_ref[...],
                         preferred_element_type=jnp.float32) * d


def _tc_mid(degp3, s1, y1, b1, W2):
    return pl.pallas_call(
        _mid_body,
        grid=(GRID,),
        in_specs=[
            pl.BlockSpec((NC, R, 1), lambda i: (0, i, 0)),
            pl.BlockSpec((NC, R, D), lambda i: (0, i, 0)),
            pl.BlockSpec((R, D), lambda i: (i, 0)),
            pl.BlockSpec((1, D), lambda i: (0, 0)),
            pl.BlockSpec((D, D), lambda i: (0, 0)),
        ],
        out_specs=pl.BlockSpec((R, D), lambda i: (i, 0)),
        out_shape=jax.ShapeDtypeStruct((N_PAD, D), jnp.float32),
    )(degp3, s1, y1, b1, W2)


def _post_body(degp_ref, s_ref, y_ref, b_ref, bt_ref, o_ref, acc, cnt):
    i = pl.program_id(0)

    @pl.when(i == 0)
    def _init():
        acc[...] = jnp.zeros_like(acc)
        cnt[...] = jnp.zeros_like(cnt)

    p = degp_ref[...]
    d = lax.rsqrt(1.0 + p[0] + p[1])
    sm = s_ref[...]
    h = jnp.maximum(d * (sm[0] + sm[1] + y_ref[...]) + b_ref[...], 0.0)
    bt = bt_ref[...]
    oh = (lax.broadcasted_iota(jnp.int32, (G, R), 0) == bt).astype(jnp.float32)
    acc[...] += jnp.dot(oh, h, preferred_element_type=jnp.float32)
    cnt[...] += jnp.sum(oh, axis=1, keepdims=True)

    @pl.when(i == GRID - 1)
    def _fin():
        o_ref[...] = acc[...] / jnp.maximum(cnt[...], 1.0)


def _tc_post(degp3, s2, y2, b2, batch_row):
    return pl.pallas_call(
        _post_body,
        grid=(GRID,),
        in_specs=[
            pl.BlockSpec((NC, R, 1), lambda i: (0, i, 0)),
            pl.BlockSpec((NC, R, D), lambda i: (0, i, 0)),
            pl.BlockSpec((R, D), lambda i: (i, 0)),
            pl.BlockSpec((1, D), lambda i: (0, 0)),
            pl.BlockSpec((1, R), lambda i: (0, i)),
        ],
        out_specs=pl.BlockSpec((G, D), lambda i: (0, 0)),
        out_shape=jax.ShapeDtypeStruct((G, D), jnp.float32),
        scratch_shapes=[
            pltpu.VMEM((G, D), jnp.float32),
            pltpu.VMEM((G, 1), jnp.float32),
        ],
    )(degp3, s2, y2, b2, batch_row)


def kernel(x, edge_index, batch, W1, b1, W2, b2):
    pad_ids = N_NODES + jnp.arange(EPT_PAD - EPT, dtype=jnp.int32)[None, :]
    pad_blk = jnp.broadcast_to(pad_ids, (NW, EPT_PAD - EPT))
    src3 = jnp.concatenate(
        [edge_index[0].reshape(NW, EPT), pad_blk], axis=1).reshape(NW, EC, KW)
    dst3 = jnp.concatenate(
        [edge_index[1].reshape(NW, EPT), pad_blk], axis=1).reshape(NW, EC, KW)

    x_pad = jnp.pad(x, ((0, N_PAD - N_NODES), (0, 0)))
    batch_row = jnp.pad(batch, (0, N_PAD - N_NODES),
                        constant_values=G).reshape(1, N_PAD)
    b1r = b1.reshape(1, D)
    b2r = b2.reshape(1, D)

    degp = _deg_kernel()(dst3)
    degp3 = degp.reshape(NC, N_PAD, 1)
    y1 = _tc_pre(degp3, x_pad, W1)
    s1 = _agg_kernel()(y1, src3, dst3)
    y2 = _tc_mid(degp3, s1, y1, b1r, W2)
    s2 = _agg_kernel()(y2, src3, dst3)
    return _tc_post(degp3, s2, y2, b2r, batch_row)
